# Initial kernel scaffold; baseline (speedup 1.0000x reference)
import jax
import jax.numpy as jnp
from jax.experimental import pallas as pl


def kernel(query, support, q_l_conn, q_l_deg, q_r_conn, q_r_deg, s_l_conn, s_l_deg, s_r_conn, s_r_deg, emb, gcn_w_W, gcn_w_b, gcn_b, se_w1, se_b1, se_w2, se_b2, se_ln_g, se_ln_b, lstm_wih, lstm_whh, lstm_bih, lstm_bhh):
    raise NotImplementedError("write your pallas kernel here")


# speedup vs baseline: 10.2019x; 10.2019x over previous
import functools

import jax
import jax.numpy as jnp
from jax.experimental import pallas as pl
from jax.experimental.pallas import tpu as pltpu

_D = 128
_K = 32
_BBLK = 512


def _mean_cat(conn, cids, emb):
    rel = emb[conn[..., 0]]
    ent = emb[conn[..., 1]]
    center = emb[cids][:, None, :]
    num = jnp.sum(center * ent, axis=-1)
    den = jnp.linalg.norm(center, axis=-1) * jnp.linalg.norm(ent, axis=-1)
    sim = num / jnp.maximum(den, 1e-8)
    _, idx = jax.lax.top_k(sim, _K)
    relm = jnp.mean(jnp.take_along_axis(rel, idx[:, :, None], axis=1), axis=1)
    entm = jnp.mean(jnp.take_along_axis(ent, idx[:, :, None], axis=1), axis=1)
    return jnp.concatenate([relm, entm], axis=-1)


def _ln(x, g, b):
    n = x.shape[-1]
    mu = jnp.mean(x, axis=-1, keepdims=True)
    var = jnp.sum((x - mu) ** 2, axis=-1, keepdims=True) / (n - 1)
    sd = jnp.sqrt(var)
    return g * (x - mu) / (sd + 1e-3) + b


def _senc(x, w1, b1, w2, b2, g, b):
    h = jax.nn.relu(jnp.dot(x, w1, preferred_element_type=jnp.float32) + b1)
    h = jnp.dot(h, w2, preferred_element_type=jnp.float32) + b2
    return _ln(h + x, g, b)


def _dense_body(mcq_ref, mcs_ref, gw_ref, gb_ref, w1_ref, b1_ref, w2_ref,
                b2_ref, lng_ref, lnb_ref, wih_ref, whh_ref, brow_ref, out_ref):
    f32 = jnp.float32
    gw = gw_ref[...]
    gb = gb_ref[...]
    w1 = w1_ref[...]
    b1 = b1_ref[...]
    w2 = w2_ref[...]
    b2 = b2_ref[...]
    lng = lng_ref[...]
    lnb = lnb_ref[...]

    mcs = mcs_ref[...]
    s_gcn = jnp.tanh(jnp.dot(mcs, gw, preferred_element_type=f32) + gb)
    sn8 = jnp.concatenate([s_gcn[0:8], s_gcn[8:16]], axis=1)
    s_enc = _senc(sn8, w1, b1, w2, b2, lng, lnb)
    smask = (jax.lax.broadcasted_iota(jnp.int32, (8, 1), 0) < 5).astype(f32)
    sg = jnp.sum(s_enc * smask, axis=0, keepdims=True) * (1.0 / 5.0)

    mcq = mcq_ref[...]
    q_l = jnp.tanh(jnp.dot(mcq[:, :256], gw, preferred_element_type=f32) + gb)
    q_r = jnp.tanh(jnp.dot(mcq[:, 256:], gw, preferred_element_type=f32) + gb)
    qn = jnp.concatenate([q_l, q_r], axis=1)
    qx = _senc(qn, w1, b1, w2, b2, lng, lnb)

    wih = wih_ref[...]
    whh = whh_ref[...]
    brow = brow_ref[...]
    whh_a = whh[:, :256]
    whh_b = whh[:, 256:]

    qwih = jax.lax.dot_general(qx, wih, (((1,), (1,)), ((), ())),
                               preferred_element_type=f32) + brow
    supw = jax.lax.dot_general(sg, whh_b, (((1,), (1,)), ((), ())),
                               preferred_element_type=f32)

    bb = qx.shape[0]
    c = jnp.zeros((bb, 512), f32)
    h = qx
    for t in range(4):
        if t == 0:
            gates = qwih
        else:
            gates = qwih + supw + jax.lax.dot_general(
                h, whh_a, (((1,), (1,)), ((), ())), preferred_element_type=f32)
        gi = gates[:, 0:512]
        gf = gates[:, 512:1024]
        gg = gates[:, 1024:1536]
        go = gates[:, 1536:2048]
        c = jax.nn.sigmoid(gf) * c + jax.nn.sigmoid(gi) * jnp.tanh(gg)
        hc = jax.nn.sigmoid(go) * jnp.tanh(c)
        h = qx + hc[:, :256]

    qf = h / jnp.maximum(
        jnp.sqrt(jnp.sum(h * h, axis=1, keepdims=True)), 1e-12)
    sgn = sg / jnp.maximum(jnp.sqrt(jnp.sum(sg * sg)), 1e-12)
    out_ref[...] = jnp.sum(qf * sgn, axis=1)


def _dense_call(mcq, mcs16, gw, gb, w1, b1, w2, b2, lng, lnb, wih, whh, brow):
    B = mcq.shape[0]
    grid = (B // _BBLK,)
    full = lambda shape: pl.BlockSpec(shape, lambda i: (0,) * len(shape))
    return pl.pallas_call(
        _dense_body,
        grid=grid,
        in_specs=[
            pl.BlockSpec((_BBLK, 512), lambda i: (i, 0)),
            full((16, 256)),
            full((256, 128)), full((128,)),
            full((256, 512)), full((512,)),
            full((512, 256)), full((256,)),
            full((256,)), full((256,)),
            full((2048, 256)), full((2048, 512)), full((2048,)),
        ],
        out_specs=pl.BlockSpec((_BBLK,), lambda i: (i,)),
        out_shape=jax.ShapeDtypeStruct((B,), jnp.float32),
    )(mcq, mcs16, gw, gb, w1, b1, w2, b2, lng, lnb, wih, whh, brow)


def kernel(query, support, q_l_conn, q_l_deg, q_r_conn, q_r_deg, s_l_conn,
           s_l_deg, s_r_conn, s_r_deg, emb, gcn_w_W, gcn_w_b, gcn_b, se_w1,
           se_b1, se_w2, se_b2, se_ln_g, se_ln_b, lstm_wih, lstm_whh,
           lstm_bih, lstm_bhh):
    mc_ql = _mean_cat(q_l_conn, query[:, 0], emb)
    mc_qr = _mean_cat(q_r_conn, query[:, 1], emb)
    mc_sl = _mean_cat(s_l_conn, support[:, 0], emb)
    mc_sr = _mean_cat(s_r_conn, support[:, 1], emb)

    mcq = jnp.concatenate([mc_ql, mc_qr], axis=1)
    mcs16 = jnp.zeros((16, 256), jnp.float32)
    mcs16 = mcs16.at[0:5].set(mc_sl).at[8:13].set(mc_sr)

    gb = gcn_w_b + gcn_b
    brow = lstm_bih + lstm_bhh
    return _dense_call(mcq, mcs16, gcn_w_W, gb, se_w1, se_b1, se_w2, se_b2,
                       se_ln_g, se_ln_b, lstm_wih, lstm_whh, brow)
